# initial kernel scaffold (unmeasured)
import jax
import jax.numpy as jnp
from jax import lax
from jax.experimental import pallas as pl
from jax.experimental.pallas import tpu as pltpu

N_DEV = 8
B, SQ, D = 4, 256, 1024
H, DH = 8, 128
SKV = 1024
R = B * SQ
SCALE = 0.08838834764831843

_sem_signal = getattr(pl, "semaphore_signal", None) or pltpu.semaphore_signal
_sem_wait = getattr(pl, "semaphore_wait", None) or pltpu.semaphore_wait
_DeviceIdType = getattr(pl, "DeviceIdType", None) or pltpu.DeviceIdType
_CompilerParams = getattr(pltpu, "CompilerParams", None) or pltpu.TPUCompilerParams


def kernel(x, Wq, Wo, K_ext, V_ext):
    def body(x_ref, wq_ref, wo_ref, k_ref, v_ref, out_ref,
             acc_ref, attn_ref, recv_ref, send_sems, recv_sems):
        p = lax.axis_index("i")
        b0 = p & 1
        b1 = (p >> 1) & 1
        b2 = (p >> 2) & 1

        wq = wq_ref[...].astype(jnp.bfloat16)
        for b in range(B):
            xb = x_ref[b].astype(jnp.bfloat16)
            qb = lax.dot_general(xb, wq, (((1,), (0,)), ((), ())),
                                 preferred_element_type=jnp.float32)
            qb = (qb * SCALE).astype(jnp.bfloat16)
            for h in range(H):
                q = qb[:, h * DH:(h + 1) * DH]
                k = k_ref[b, :, h, :].astype(jnp.bfloat16)
                v = v_ref[b, :, h, :].astype(jnp.bfloat16)
                s = lax.dot_general(q, k, (((1,), (1,)), ((), ())),
                                    preferred_element_type=jnp.float32)
                m = jnp.max(s, axis=-1, keepdims=True)
                e = jnp.exp(s - m)
                l = jnp.sum(e, axis=-1, keepdims=True)
                pr = (e / l).astype(jnp.bfloat16)
                o = lax.dot_general(pr, v, (((1,), (0,)), ((), ())),
                                    preferred_element_type=jnp.float32)
                attn_ref[b * SQ:(b + 1) * SQ, h * DH:(h + 1) * DH] = (
                    o.astype(jnp.bfloat16))
        wo = wo_ref[...].astype(jnp.bfloat16)
        acc_ref[...] = lax.dot_general(
            attn_ref[...], wo, (((1,), (0,)), ((), ())),
            preferred_element_type=jnp.float32)

        barrier = pltpu.get_barrier_semaphore()
        for mask in (1, 3, 4):
            _sem_signal(barrier, inc=1, device_id=(p ^ mask,),
                        device_id_type=_DeviceIdType.MESH)
        _sem_wait(barrier, 3)

        a1 = 512 * b2
        a2 = a1 + 256 * b1
        a3 = a2 + 128 * b0

        rs_steps = [
            (4, 512 * (1 - b2), 512, a1, 0),
            (3, a1 + 256 * (1 - b1), 256, a2, 1),
            (1, a2 + 128 * (1 - b0), 128, a3, 0),
        ]
        for step, (mask, send_start, size, keep_start, slot) in enumerate(rs_steps):
            rdma = pltpu.make_async_remote_copy(
                src_ref=acc_ref.at[pl.ds(send_start, size)],
                dst_ref=recv_ref.at[slot, pl.ds(0, size)],
                send_sem=send_sems.at[step],
                recv_sem=recv_sems.at[step],
                device_id=(p ^ mask,),
                device_id_type=_DeviceIdType.MESH,
            )
            rdma.start()
            rdma.wait()
            acc_ref[pl.ds(keep_start, size)] = (
                acc_ref[pl.ds(keep_start, size)]
                + recv_ref[slot, pl.ds(0, size)])

        ag_steps = [(1, a3, 128, 3), (3, a2, 256, 4), (4, a1, 512, 5)]
        for mask, start, size, step in ag_steps:
            rdma = pltpu.make_async_remote_copy(
                src_ref=acc_ref.at[pl.ds(start, size)],
                dst_ref=acc_ref.at[pl.ds(start, size)],
                send_sem=send_sems.at[step],
                recv_sem=recv_sems.at[step],
                device_id=(p ^ mask,),
                device_id_type=_DeviceIdType.MESH,
            )
            rdma.start()
            rdma.wait()

        for b in range(B):
            out_ref[b] = acc_ref[b * SQ:(b + 1) * SQ, :]

    return pl.pallas_call(
        body,
        out_shape=jax.ShapeDtypeStruct((B, SQ, D), jnp.float32),
        in_specs=[pl.BlockSpec(memory_space=pltpu.VMEM)] * 5,
        out_specs=pl.BlockSpec(memory_space=pltpu.VMEM),
        scratch_shapes=[
            pltpu.VMEM((R, D), jnp.float32),
            pltpu.VMEM((R, D), jnp.bfloat16),
            pltpu.VMEM((2, 512, D), jnp.float32),
            pltpu.SemaphoreType.DMA((6,)),
            pltpu.SemaphoreType.DMA((6,)),
        ],
        compiler_params=_CompilerParams(collective_id=0),
    )(x, Wq, Wo, K_ext, V_ext)


# baseline (device time: 175334 ns/iter reference)
import jax
import jax.numpy as jnp
from jax import lax
from jax.experimental import pallas as pl
from jax.experimental.pallas import tpu as pltpu

N_DEV = 8
B, SQ, D = 4, 256, 1024
H, DH = 8, 128
SKV = 1024
R = B * SQ
SCALE = 0.08838834764831843

_sem_signal = getattr(pl, "semaphore_signal", None) or pltpu.semaphore_signal
_sem_wait = getattr(pl, "semaphore_wait", None) or pltpu.semaphore_wait
_DeviceIdType = getattr(pl, "DeviceIdType", None) or pltpu.DeviceIdType
_CompilerParams = getattr(pltpu, "CompilerParams", None) or pltpu.TPUCompilerParams


def kernel(x, Wq, Wo, K_ext, V_ext):
    def body(x_ref, wq_ref, wo_ref, k_ref, v_ref, out_ref,
             acc_ref, attn_ref, recv_ref, send_sems, recv_sems):
        p = lax.axis_index("i")
        b0 = p & 1
        b1 = (p >> 1) & 1
        b2 = (p >> 2) & 1

        wq = wq_ref[...].astype(jnp.bfloat16)
        for b in range(B):
            xb = x_ref[b].astype(jnp.bfloat16)
            qb = lax.dot_general(xb, wq, (((1,), (0,)), ((), ())),
                                 preferred_element_type=jnp.float32)
            qb = (qb * SCALE).astype(jnp.bfloat16)
            for h in range(H):
                q = qb[:, h * DH:(h + 1) * DH]
                k = k_ref[b, :, h, :].astype(jnp.bfloat16)
                v = v_ref[b, :, h, :].astype(jnp.bfloat16)
                s = lax.dot_general(q, k, (((1,), (1,)), ((), ())),
                                    preferred_element_type=jnp.float32)
                m = jnp.max(s, axis=-1, keepdims=True)
                e = jnp.exp(s - m)
                l = jnp.sum(e, axis=-1, keepdims=True)
                pr = (e / l).astype(jnp.bfloat16)
                o = lax.dot_general(pr, v, (((1,), (0,)), ((), ())),
                                    preferred_element_type=jnp.float32)
                attn_ref[b * SQ:(b + 1) * SQ, h * DH:(h + 1) * DH] = (
                    o.astype(jnp.bfloat16))
        wo = wo_ref[...].astype(jnp.bfloat16)
        acc_ref[...] = lax.dot_general(
            attn_ref[...], wo, (((1,), (0,)), ((), ())),
            preferred_element_type=jnp.float32)

        barrier = pltpu.get_barrier_semaphore()
        for mask in (1, 3, 4):
            _sem_signal(barrier, inc=1, device_id=(p ^ mask,),
                        device_id_type=_DeviceIdType.MESH)
        _sem_wait(barrier, 3)

        a1 = 512 * b2
        a2 = a1 + 256 * b1
        a3 = a2 + 128 * b0

        rs_steps = [
            (4, 512 * (1 - b2), 512, a1, 0),
            (3, a1 + 256 * (1 - b1), 256, a2, 1),
            (1, a2 + 128 * (1 - b0), 128, a3, 0),
        ]
        for step, (mask, send_start, size, keep_start, slot) in enumerate(rs_steps):
            rdma = pltpu.make_async_remote_copy(
                src_ref=acc_ref.at[pl.ds(send_start, size)],
                dst_ref=recv_ref.at[slot, pl.ds(0, size)],
                send_sem=send_sems.at[step],
                recv_sem=recv_sems.at[step],
                device_id=(p ^ mask,),
                device_id_type=_DeviceIdType.MESH,
            )
            rdma.start()
            rdma.wait()
            acc_ref[pl.ds(keep_start, size)] = (
                acc_ref[pl.ds(keep_start, size)]
                + recv_ref[slot, pl.ds(0, size)])

        ag_steps = [(1, a3, 128, 3), (3, a2, 256, 4), (4, a1, 512, 5)]
        for mask, start, size, step in ag_steps:
            rdma = pltpu.make_async_remote_copy(
                src_ref=acc_ref.at[pl.ds(start, size)],
                dst_ref=acc_ref.at[pl.ds(start, size)],
                send_sem=send_sems.at[step],
                recv_sem=recv_sems.at[step],
                device_id=(p ^ mask,),
                device_id_type=_DeviceIdType.MESH,
            )
            rdma.start()
            rdma.wait()

        for b in range(B):
            out_ref[b] = acc_ref[b * SQ:(b + 1) * SQ, :]

    return pl.pallas_call(
        body,
        out_shape=jax.ShapeDtypeStruct((B, SQ, D), jnp.float32),
        in_specs=[pl.BlockSpec(memory_space=pltpu.VMEM)] * 5,
        out_specs=pl.BlockSpec(memory_space=pltpu.VMEM),
        scratch_shapes=[
            pltpu.VMEM((R, D), jnp.float32),
            pltpu.VMEM((R, D), jnp.bfloat16),
            pltpu.VMEM((2, 512, D), jnp.float32),
            pltpu.SemaphoreType.DMA((6,)),
            pltpu.SemaphoreType.DMA((6,)),
        ],
        compiler_params=_CompilerParams(
            collective_id=0, vmem_limit_bytes=100 * 1024 * 1024),
    )(x, Wq, Wo, K_ext, V_ext)


# device time: 134679 ns/iter; 1.3019x vs baseline; 1.3019x over previous
import jax
import jax.numpy as jnp
from jax import lax
from jax.experimental import pallas as pl
from jax.experimental.pallas import tpu as pltpu

N_DEV = 8
B, SQ, D = 4, 256, 1024
H, DH = 8, 128
SKV = 1024
R = B * SQ
SCALE = 0.08838834764831843

_sem_signal = getattr(pl, "semaphore_signal", None) or pltpu.semaphore_signal
_sem_wait = getattr(pl, "semaphore_wait", None) or pltpu.semaphore_wait
_DeviceIdType = getattr(pl, "DeviceIdType", None) or pltpu.DeviceIdType
_CompilerParams = getattr(pltpu, "CompilerParams", None) or pltpu.TPUCompilerParams


def kernel(x, Wq, Wo, K_ext, V_ext):
    def body(x_ref, wq_ref, wo_ref, k_ref, v_ref, out_ref,
             acc_ref, attn_ref, recv_ref, send_sems, recv_sems):
        p = lax.axis_index("i")
        b0 = p & 1
        b1 = (p >> 1) & 1
        b2 = (p >> 2) & 1

        wq = wq_ref[...].astype(jnp.bfloat16)
        for b in range(B):
            xb = x_ref[b].astype(jnp.bfloat16)
            qb = lax.dot_general(xb, wq, (((1,), (0,)), ((), ())),
                                 preferred_element_type=jnp.float32)
            qb = (qb * SCALE).astype(jnp.bfloat16)
            for h in range(H):
                q = qb[:, h * DH:(h + 1) * DH]
                k = k_ref[b, :, h, :].astype(jnp.bfloat16)
                v = v_ref[b, :, h, :].astype(jnp.bfloat16)
                s = lax.dot_general(q, k, (((1,), (1,)), ((), ())),
                                    preferred_element_type=jnp.float32)
                m = jnp.max(s, axis=-1, keepdims=True)
                e = jnp.exp(s - m)
                l = jnp.sum(e, axis=-1, keepdims=True)
                pr = (e / l).astype(jnp.bfloat16)
                o = lax.dot_general(pr, v, (((1,), (0,)), ((), ())),
                                    preferred_element_type=jnp.float32)
                attn_ref[b * SQ:(b + 1) * SQ, h * DH:(h + 1) * DH] = (
                    o.astype(jnp.bfloat16))
        wo = wo_ref[...].astype(jnp.bfloat16)
        acc_ref[...] = lax.dot_general(
            attn_ref[...], wo, (((1,), (0,)), ((), ())),
            preferred_element_type=jnp.float32).astype(jnp.bfloat16)

        barrier = pltpu.get_barrier_semaphore()
        for mask in (1, 3, 4):
            _sem_signal(barrier, inc=1, device_id=(p ^ mask,),
                        device_id_type=_DeviceIdType.MESH)
        _sem_wait(barrier, 3)

        a1 = 512 * b2
        a2 = a1 + 256 * b1
        a3 = a2 + 128 * b0

        rs_steps = [
            (4, 512 * (1 - b2), 512, a1, 0),
            (3, a1 + 256 * (1 - b1), 256, a2, 1),
            (1, a2 + 128 * (1 - b0), 128, a3, 0),
        ]
        for step, (mask, send_start, size, keep_start, slot) in enumerate(rs_steps):
            rdma = pltpu.make_async_remote_copy(
                src_ref=acc_ref.at[pl.ds(send_start, size)],
                dst_ref=recv_ref.at[slot, pl.ds(0, size)],
                send_sem=send_sems.at[step],
                recv_sem=recv_sems.at[step],
                device_id=(p ^ mask,),
                device_id_type=_DeviceIdType.MESH,
            )
            rdma.start()
            rdma.wait()
            acc_ref[pl.ds(keep_start, size)] = (
                acc_ref[pl.ds(keep_start, size)].astype(jnp.float32)
                + recv_ref[slot, pl.ds(0, size)].astype(jnp.float32)
            ).astype(jnp.bfloat16)

        ag_steps = [(1, a3, 128, 3), (3, a2, 256, 4), (4, a1, 512, 5)]
        for mask, start, size, step in ag_steps:
            rdma = pltpu.make_async_remote_copy(
                src_ref=acc_ref.at[pl.ds(start, size)],
                dst_ref=acc_ref.at[pl.ds(start, size)],
                send_sem=send_sems.at[step],
                recv_sem=recv_sems.at[step],
                device_id=(p ^ mask,),
                device_id_type=_DeviceIdType.MESH,
            )
            rdma.start()
            rdma.wait()

        for b in range(B):
            out_ref[b] = acc_ref[b * SQ:(b + 1) * SQ, :].astype(jnp.float32)

    return pl.pallas_call(
        body,
        out_shape=jax.ShapeDtypeStruct((B, SQ, D), jnp.float32),
        in_specs=[pl.BlockSpec(memory_space=pltpu.VMEM)] * 5,
        out_specs=pl.BlockSpec(memory_space=pltpu.VMEM),
        scratch_shapes=[
            pltpu.VMEM((R, D), jnp.bfloat16),
            pltpu.VMEM((R, D), jnp.bfloat16),
            pltpu.VMEM((2, 512, D), jnp.bfloat16),
            pltpu.SemaphoreType.DMA((6,)),
            pltpu.SemaphoreType.DMA((6,)),
        ],
        compiler_params=_CompilerParams(
            collective_id=0, vmem_limit_bytes=100 * 1024 * 1024),
    )(x, Wq, Wo, K_ext, V_ext)


# device time: 112098 ns/iter; 1.5641x vs baseline; 1.2014x over previous
import jax
import jax.numpy as jnp
from jax import lax
from jax.experimental import pallas as pl
from jax.experimental.pallas import tpu as pltpu

N_DEV = 8
B, SQ, D = 4, 256, 1024
H, DH = 8, 128
SKV = 1024
R = B * SQ
SCALE = 0.08838834764831843
COMM = True

_sem_signal = getattr(pl, "semaphore_signal", None) or pltpu.semaphore_signal
_sem_wait = getattr(pl, "semaphore_wait", None) or pltpu.semaphore_wait
_DeviceIdType = getattr(pl, "DeviceIdType", None) or pltpu.DeviceIdType
_CompilerParams = getattr(pltpu, "CompilerParams", None) or pltpu.TPUCompilerParams

F32 = jnp.float32
BF16 = jnp.bfloat16


def kernel(x, Wq, Wo, K_ext, V_ext):
    def body(x_ref, wq_ref, wo_ref, k_ref, v_ref, out_ref,
             q_ref, attn_ref, recv_ref, send_sems, recv_sems):
        p = lax.axis_index("i")
        b0 = p & 1
        b1 = (p >> 1) & 1
        b2 = (p >> 2) & 1

        def exchange(step, mask, src, dst):
            return pltpu.make_async_remote_copy(
                src_ref=src, dst_ref=dst,
                send_sem=send_sems.at[step], recv_sem=recv_sems.at[step],
                device_id=(p ^ mask,), device_id_type=_DeviceIdType.MESH,
            )

        if COMM:
            barrier = pltpu.get_barrier_semaphore()
            for mask in (1, 3, 4):
                _sem_signal(barrier, inc=1, device_id=(p ^ mask,),
                            device_id_type=_DeviceIdType.MESH)
            _sem_wait(barrier, 3)

        a1 = 512 * b2
        a2 = a1 + 256 * b1
        a3 = a2 + 128 * b0
        send1 = 512 * (1 - b2)

        wq = wq_ref[...].astype(BF16)
        xf = x_ref[...].reshape(R, D).astype(BF16)
        q_ref[...] = (
            lax.dot_general(xf, wq, (((1,), (0,)), ((), ())),
                            preferred_element_type=F32) * SCALE
        ).astype(BF16)

        wo = wo_ref[...].astype(BF16)

        rs1 = None
        for j in range(B):
            b = jnp.int32(j) ^ (2 * (1 - b2))
            row = b * SQ
            for h in range(H):
                q = q_ref[pl.ds(row, SQ), h * DH:(h + 1) * DH]
                k = k_ref[b, :, h, :].astype(BF16)
                v = v_ref[b, :, h, :].astype(BF16)
                s = lax.dot_general(q, k, (((1,), (1,)), ((), ())),
                                    preferred_element_type=F32)
                e = jnp.exp(s)
                l = jnp.sum(e, axis=-1, keepdims=True)
                o = lax.dot_general(e.astype(BF16), v,
                                    (((1,), (0,)), ((), ())),
                                    preferred_element_type=F32)
                attn_ref[pl.ds(row, SQ), h * DH:(h + 1) * DH] = (
                    (o / l).astype(BF16))
            if j == 1 and COMM:
                out_ref[pl.ds(send1, 512)] = lax.dot_general(
                    attn_ref[pl.ds(send1, 512), :], wo,
                    (((1,), (0,)), ((), ())),
                    preferred_element_type=F32).astype(BF16)
                rs1 = exchange(0, 4, out_ref.at[pl.ds(send1, 512)],
                               recv_ref.at[0, pl.ds(0, 512)])
                rs1.start()

        out_ref[pl.ds(a1, 512)] = lax.dot_general(
            attn_ref[pl.ds(a1, 512), :], wo, (((1,), (0,)), ((), ())),
            preferred_element_type=F32).astype(BF16)

        if not COMM:
            out_ref[pl.ds(send1, 512)] = lax.dot_general(
                attn_ref[pl.ds(send1, 512), :], wo,
                (((1,), (0,)), ((), ())),
                preferred_element_type=F32).astype(BF16)
            return

        def add_into(start, size, slot):
            out_ref[pl.ds(start, size)] = (
                out_ref[pl.ds(start, size)].astype(F32)
                + recv_ref[slot, pl.ds(0, size)].astype(F32)
            ).astype(BF16)

        rs1.wait_recv()
        add_into(a1, 512, 0)

        rs2 = exchange(1, 3, out_ref.at[pl.ds(a1 + 256 * (1 - b1), 256)],
                       recv_ref.at[1, pl.ds(0, 256)])
        rs2.start()
        rs2.wait_recv()
        add_into(a2, 256, 1)

        rs3 = exchange(2, 1, out_ref.at[pl.ds(a2 + 128 * (1 - b0), 128)],
                       recv_ref.at[0, pl.ds(0, 128)])
        rs3.start()
        rs3.wait_recv()
        add_into(a3, 128, 0)

        ag4 = exchange(3, 1, out_ref.at[pl.ds(a3, 128)],
                       out_ref.at[pl.ds(a3, 128)])
        ag4.start()
        ag4.wait_recv()
        ag5 = exchange(4, 3, out_ref.at[pl.ds(a2, 256)],
                       out_ref.at[pl.ds(a2, 256)])
        ag5.start()
        ag5.wait_recv()
        ag6 = exchange(5, 4, out_ref.at[pl.ds(a1, 512)],
                       out_ref.at[pl.ds(a1, 512)])
        ag6.start()
        ag6.wait_recv()

        for r in (rs1, rs2, rs3, ag4, ag5, ag6):
            r.wait_send()

    out = pl.pallas_call(
        body,
        out_shape=jax.ShapeDtypeStruct((R, D), BF16),
        in_specs=[pl.BlockSpec(memory_space=pltpu.VMEM)] * 5,
        out_specs=pl.BlockSpec(memory_space=pltpu.VMEM),
        scratch_shapes=[
            pltpu.VMEM((R, D), BF16),
            pltpu.VMEM((R, D), BF16),
            pltpu.VMEM((2, 512, D), BF16),
            pltpu.SemaphoreType.DMA((6,)),
            pltpu.SemaphoreType.DMA((6,)),
        ],
        compiler_params=_CompilerParams(
            vmem_limit_bytes=100 * 1024 * 1024,
            **({"collective_id": 0} if COMM else {})),
    )(x, Wq, Wo, K_ext, V_ext)
    return out.reshape(B, SQ, D).astype(F32)


# device time: 106952 ns/iter; 1.6394x vs baseline; 1.0481x over previous
import jax
import jax.numpy as jnp
from jax import lax
from jax.experimental import pallas as pl
from jax.experimental.pallas import tpu as pltpu

N_DEV = 8
B, SQ, D = 4, 256, 1024
H, DH = 8, 128
SKV = 1024
R = B * SQ
SCALE = 0.08838834764831843
COMM = True

_sem_signal = getattr(pl, "semaphore_signal", None) or pltpu.semaphore_signal
_sem_wait = getattr(pl, "semaphore_wait", None) or pltpu.semaphore_wait
_DeviceIdType = getattr(pl, "DeviceIdType", None) or pltpu.DeviceIdType
_CompilerParams = getattr(pltpu, "CompilerParams", None) or pltpu.TPUCompilerParams

F32 = jnp.float32
BF16 = jnp.bfloat16


def kernel(x, Wq, Wo, K_ext, V_ext):
    def body(x_ref, wq_ref, wo_ref, k_ref, v_ref, out_ref,
             q_ref, attn_ref, recv_ref, send_sems, recv_sems):
        p = lax.axis_index("i")
        b0 = p & 1
        b1 = (p >> 1) & 1
        b2 = (p >> 2) & 1

        def exchange(step, mask, src, dst):
            return pltpu.make_async_remote_copy(
                src_ref=src, dst_ref=dst,
                send_sem=send_sems.at[step], recv_sem=recv_sems.at[step],
                device_id=(p ^ mask,), device_id_type=_DeviceIdType.MESH,
            )

        if COMM:
            barrier = pltpu.get_barrier_semaphore()
            for mask in (1, 3, 4):
                _sem_signal(barrier, inc=1, device_id=(p ^ mask,),
                            device_id_type=_DeviceIdType.MESH)
            _sem_wait(barrier, 3)

        a1 = 512 * b2
        a2 = a1 + 256 * b1
        a3 = a2 + 128 * b0
        send1 = 512 * (1 - b2)

        wq = wq_ref[...].astype(BF16)
        xf = x_ref[...].reshape(R, D).astype(BF16)
        q_ref[...] = (
            lax.dot_general(xf, wq, (((1,), (0,)), ((), ())),
                            preferred_element_type=F32) * SCALE
        ).astype(BF16)

        wo = wo_ref[...].astype(BF16)

        def project(start, size):
            out_ref[pl.ds(start, size)] = lax.dot_general(
                attn_ref[pl.ds(start, size), :], wo,
                (((1,), (0,)), ((), ())),
                preferred_element_type=F32).astype(BF16)

        rs1 = rs2 = None
        j2_start = a1 + 256 * (1 - b1)
        for j in range(B):
            if j < 2:
                b = jnp.int32(j) + 2 * (1 - b2)
            elif j == 2:
                b = j2_start // SQ
            else:
                b = a2 // SQ
            row = b * SQ
            for h in range(H):
                q = q_ref[pl.ds(row, SQ), h * DH:(h + 1) * DH]
                k = k_ref[b, :, h, :].astype(BF16)
                v = v_ref[b, :, h, :].astype(BF16)
                s = lax.dot_general(q, k, (((1,), (1,)), ((), ())),
                                    preferred_element_type=F32)
                e = jnp.exp(s)
                l = jnp.sum(e, axis=-1, keepdims=True)
                o = lax.dot_general(e.astype(BF16), v,
                                    (((1,), (0,)), ((), ())),
                                    preferred_element_type=F32)
                attn_ref[pl.ds(row, SQ), h * DH:(h + 1) * DH] = (
                    (o / l).astype(BF16))
            if not COMM:
                continue
            if j == 1:
                project(send1, 512)
                rs1 = exchange(0, 4, out_ref.at[pl.ds(send1, 512)],
                               recv_ref.at[0, pl.ds(0, 512)])
                rs1.start()
            elif j == 2:
                project(j2_start, 256)
                rs1.wait_recv()
                out_ref[pl.ds(j2_start, 256)] = (
                    out_ref[pl.ds(j2_start, 256)].astype(F32)
                    + recv_ref[0, pl.ds(256 * (1 - b1), 256)].astype(F32)
                ).astype(BF16)
                rs2 = exchange(1, 3, out_ref.at[pl.ds(j2_start, 256)],
                               recv_ref.at[1, pl.ds(0, 256)])
                rs2.start()

        if not COMM:
            project(a1, 512)
            project(send1, 512)
            return

        project(a2, 256)
        rs2.wait_recv()
        out_ref[pl.ds(a2, 256)] = (
            out_ref[pl.ds(a2, 256)].astype(F32)
            + recv_ref[0, pl.ds(256 * b1, 256)].astype(F32)
            + recv_ref[1, pl.ds(0, 256)].astype(F32)
        ).astype(BF16)

        rs3 = exchange(2, 1, out_ref.at[pl.ds(a2 + 128 * (1 - b0), 128)],
                       recv_ref.at[1, pl.ds(256, 128)])
        rs3.start()
        rs3.wait_recv()
        out_ref[pl.ds(a3, 128)] = (
            out_ref[pl.ds(a3, 128)].astype(F32)
            + recv_ref[1, pl.ds(256, 128)].astype(F32)
        ).astype(BF16)

        ag4 = exchange(3, 1, out_ref.at[pl.ds(a3, 128)],
                       out_ref.at[pl.ds(a3, 128)])
        ag4.start()
        ag4.wait_recv()
        ag5 = exchange(4, 3, out_ref.at[pl.ds(a2, 256)],
                       out_ref.at[pl.ds(a2, 256)])
        ag5.start()
        ag5.wait_recv()
        ag6 = exchange(5, 4, out_ref.at[pl.ds(a1, 512)],
                       out_ref.at[pl.ds(a1, 512)])
        ag6.start()
        ag6.wait_recv()

        for r in (rs1, rs2, rs3, ag4, ag5, ag6):
            r.wait_send()

    out = pl.pallas_call(
        body,
        out_shape=jax.ShapeDtypeStruct((R, D), BF16),
        in_specs=[pl.BlockSpec(memory_space=pltpu.VMEM)] * 5,
        out_specs=pl.BlockSpec(memory_space=pltpu.VMEM),
        scratch_shapes=[
            pltpu.VMEM((R, D), BF16),
            pltpu.VMEM((R, D), BF16),
            pltpu.VMEM((2, 512, D), BF16),
            pltpu.SemaphoreType.DMA((6,)),
            pltpu.SemaphoreType.DMA((6,)),
        ],
        compiler_params=_CompilerParams(
            vmem_limit_bytes=100 * 1024 * 1024,
            **({"collective_id": 0} if COMM else {})),
    )(x, Wq, Wo, K_ext, V_ext)
    return out.reshape(B, SQ, D).astype(F32)


# device time: 81856 ns/iter; 2.1420x vs baseline; 1.3066x over previous
import jax
import jax.numpy as jnp
from jax import lax
from jax.experimental import pallas as pl
from jax.experimental.pallas import tpu as pltpu

N_DEV = 8
B, SQ, D = 4, 256, 1024
H, DH = 8, 128
SKV = 1024
R = B * SQ
SCALE = 0.08838834764831843
COMM = True

_sem_signal = getattr(pl, "semaphore_signal", None) or pltpu.semaphore_signal
_sem_wait = getattr(pl, "semaphore_wait", None) or pltpu.semaphore_wait
_DeviceIdType = getattr(pl, "DeviceIdType", None) or pltpu.DeviceIdType
_CompilerParams = getattr(pltpu, "CompilerParams", None) or pltpu.TPUCompilerParams

F32 = jnp.float32
BF16 = jnp.bfloat16


def kernel(x, Wq, Wo, K_ext, V_ext):
    def body(x_ref, wq_ref, wo_ref, k_ref, v_ref, out_ref,
             q_ref, attn_ref, recv_ref, kbuf, vbuf,
             send_sems, recv_sems, ksem, vsem):
        p = lax.axis_index("i")
        b0 = p & 1
        b1 = (p >> 1) & 1
        b2 = (p >> 2) & 1

        def exchange(step, mask, src, dst):
            return pltpu.make_async_remote_copy(
                src_ref=src, dst_ref=dst,
                send_sem=send_sems.at[step], recv_sem=recv_sems.at[step],
                device_id=(p ^ mask,), device_id_type=_DeviceIdType.MESH,
            )

        if COMM:
            barrier = pltpu.get_barrier_semaphore()
            for mask in (1, 3, 4):
                _sem_signal(barrier, inc=1, device_id=(p ^ mask,),
                            device_id_type=_DeviceIdType.MESH)
            _sem_wait(barrier, 3)

        a1 = 512 * b2
        a2 = a1 + 256 * b1
        a3 = a2 + 128 * b0
        send1 = 512 * (1 - b2)
        j2_start = a1 + 256 * (1 - b1)

        blocks = [2 * (1 - b2), 2 * (1 - b2) + 1, j2_start // SQ, a2 // SQ]

        def kv_copies(n, slot):
            j, h = divmod(n, H)
            b = blocks[j]
            return (
                pltpu.make_async_copy(k_ref.at[b, :, h, :], kbuf.at[slot],
                                      ksem.at[slot]),
                pltpu.make_async_copy(v_ref.at[b, :, h, :], vbuf.at[slot],
                                      vsem.at[slot]),
            )

        ck, cv = kv_copies(0, 0)
        ck.start()
        cv.start()

        wq = wq_ref[...].astype(BF16)
        xf = x_ref[...].reshape(R, D).astype(BF16)
        q_ref[...] = (
            lax.dot_general(xf, wq, (((1,), (0,)), ((), ())),
                            preferred_element_type=F32) * SCALE
        ).astype(BF16)

        wo = wo_ref[...].astype(BF16)

        def project(start, size):
            out_ref[pl.ds(start, size)] = lax.dot_general(
                attn_ref[pl.ds(start, size), :], wo,
                (((1,), (0,)), ((), ())),
                preferred_element_type=F32).astype(BF16)

        rs1 = rs2 = None
        for j in range(B):
            row = blocks[j] * SQ
            for h in range(H):
                n = j * H + h
                slot = n & 1
                if n + 1 < B * H:
                    nk, nv = kv_copies(n + 1, (n + 1) & 1)
                    nk.start()
                    nv.start()
                ck.wait()
                cv.wait()
                if n + 1 < B * H:
                    ck, cv = nk, nv
                k = kbuf[slot].astype(BF16)
                v = vbuf[slot].astype(BF16)
                q = q_ref[pl.ds(row, SQ), h * DH:(h + 1) * DH]
                s = lax.dot_general(q, k, (((1,), (1,)), ((), ())),
                                    preferred_element_type=F32)
                e = jnp.exp(s)
                l = jnp.sum(e, axis=-1, keepdims=True)
                o = lax.dot_general(e.astype(BF16), v,
                                    (((1,), (0,)), ((), ())),
                                    preferred_element_type=F32)
                attn_ref[pl.ds(row, SQ), h * DH:(h + 1) * DH] = (
                    (o / l).astype(BF16))
            if not COMM:
                continue
            if j == 1:
                project(send1, 512)
                rs1 = exchange(0, 4, out_ref.at[pl.ds(send1, 512)],
                               recv_ref.at[0, pl.ds(0, 512)])
                rs1.start()
            elif j == 2:
                project(j2_start, 256)
                rs1.wait_recv()
                out_ref[pl.ds(j2_start, 256)] = (
                    out_ref[pl.ds(j2_start, 256)].astype(F32)
                    + recv_ref[0, pl.ds(256 * (1 - b1), 256)].astype(F32)
                ).astype(BF16)
                rs2 = exchange(1, 3, out_ref.at[pl.ds(j2_start, 256)],
                               recv_ref.at[1, pl.ds(0, 256)])
                rs2.start()

        if not COMM:
            project(a1, 512)
            project(send1, 512)
            return

        project(a2, 256)
        rs2.wait_recv()
        out_ref[pl.ds(a2, 256)] = (
            out_ref[pl.ds(a2, 256)].astype(F32)
            + recv_ref[0, pl.ds(256 * b1, 256)].astype(F32)
            + recv_ref[1, pl.ds(0, 256)].astype(F32)
        ).astype(BF16)

        rs3 = exchange(2, 1, out_ref.at[pl.ds(a2 + 128 * (1 - b0), 128)],
                       recv_ref.at[1, pl.ds(256, 128)])
        rs3.start()
        rs3.wait_recv()
        out_ref[pl.ds(a3, 128)] = (
            out_ref[pl.ds(a3, 128)].astype(F32)
            + recv_ref[1, pl.ds(256, 128)].astype(F32)
        ).astype(BF16)

        ag4 = exchange(3, 1, out_ref.at[pl.ds(a3, 128)],
                       out_ref.at[pl.ds(a3, 128)])
        ag4.start()
        ag4.wait_recv()
        ag5 = exchange(4, 3, out_ref.at[pl.ds(a2, 256)],
                       out_ref.at[pl.ds(a2, 256)])
        ag5.start()
        ag5.wait_recv()
        ag6 = exchange(5, 4, out_ref.at[pl.ds(a1, 512)],
                       out_ref.at[pl.ds(a1, 512)])
        ag6.start()
        ag6.wait_recv()

        for r in (rs1, rs2, rs3, ag4, ag5, ag6):
            r.wait_send()

    out = pl.pallas_call(
        body,
        out_shape=jax.ShapeDtypeStruct((R, D), BF16),
        in_specs=[pl.BlockSpec(memory_space=pltpu.VMEM)] * 3
        + [pl.BlockSpec(memory_space=pl.ANY)] * 2,
        out_specs=pl.BlockSpec(memory_space=pltpu.VMEM),
        scratch_shapes=[
            pltpu.VMEM((R, D), BF16),
            pltpu.VMEM((R, D), BF16),
            pltpu.VMEM((2, 512, D), BF16),
            pltpu.VMEM((2, SKV, DH), F32),
            pltpu.VMEM((2, SKV, DH), F32),
            pltpu.SemaphoreType.DMA((6,)),
            pltpu.SemaphoreType.DMA((6,)),
            pltpu.SemaphoreType.DMA((2,)),
            pltpu.SemaphoreType.DMA((2,)),
        ],
        compiler_params=_CompilerParams(
            vmem_limit_bytes=100 * 1024 * 1024,
            **({"collective_id": 0} if COMM else {})),
    )(x, Wq, Wo, K_ext, V_ext)
    return out.reshape(B, SQ, D).astype(F32)


# device time: 66693 ns/iter; 2.6290x vs baseline; 1.2274x over previous
import jax
import jax.numpy as jnp
from jax import lax
from jax.experimental import pallas as pl
from jax.experimental.pallas import tpu as pltpu

N_DEV = 8
B, SQ, D = 4, 256, 1024
H, DH = 8, 128
SKV = 1024
R = B * SQ
SCALE = 0.08838834764831843
COMM = True

_sem_signal = getattr(pl, "semaphore_signal", None) or pltpu.semaphore_signal
_sem_wait = getattr(pl, "semaphore_wait", None) or pltpu.semaphore_wait
_DeviceIdType = getattr(pl, "DeviceIdType", None) or pltpu.DeviceIdType
_CompilerParams = getattr(pltpu, "CompilerParams", None) or pltpu.TPUCompilerParams

F32 = jnp.float32
BF16 = jnp.bfloat16


def kernel(x, Wq, Wo, K_ext, V_ext):
    def body(x_ref, wq_ref, wo_ref, k_ref, v_ref, out_ref,
             q_ref, attn_ref, recv_ref, kbuf, vbuf,
             send_sems, recv_sems, ksem, vsem):
        p = lax.axis_index("i")
        b0 = p & 1
        b1 = (p >> 1) & 1
        b2 = (p >> 2) & 1

        def exchange(step, mask, src, dst):
            return pltpu.make_async_remote_copy(
                src_ref=src, dst_ref=dst,
                send_sem=send_sems.at[step], recv_sem=recv_sems.at[step],
                device_id=(p ^ mask,), device_id_type=_DeviceIdType.MESH,
            )

        if COMM:
            barrier = pltpu.get_barrier_semaphore()
            for mask in (1, 3, 4):
                _sem_signal(barrier, inc=1, device_id=(p ^ mask,),
                            device_id_type=_DeviceIdType.MESH)
            _sem_wait(barrier, 3)

        a1 = 512 * b2
        a2 = a1 + 256 * b1
        a3 = a2 + 128 * b0
        send1 = 512 * (1 - b2)
        j2_start = a1 + 256 * (1 - b1)

        blocks = [2 * (1 - b2), 2 * (1 - b2) + 1, j2_start // SQ, a2 // SQ]

        def kv_copies(n, slot):
            j, h = divmod(n, H)
            b = blocks[j]
            return (
                pltpu.make_async_copy(k_ref.at[b, :, h, :], kbuf.at[slot],
                                      ksem.at[slot]),
                pltpu.make_async_copy(v_ref.at[b, :, h, :], vbuf.at[slot],
                                      vsem.at[slot]),
            )

        DEPTH = 3
        inflight = {}
        for n in range(DEPTH):
            inflight[n] = kv_copies(n, n & 3)
            inflight[n][0].start()
            inflight[n][1].start()

        wq = wq_ref[...].astype(BF16)
        xf = x_ref[...].reshape(R, D).astype(BF16)
        q_ref[...] = (
            lax.dot_general(xf, wq, (((1,), (0,)), ((), ())),
                            preferred_element_type=F32) * SCALE
        ).astype(BF16)

        wo = wo_ref[...].astype(BF16)

        def project(start, size):
            out_ref[pl.ds(start, size)] = lax.dot_general(
                attn_ref[pl.ds(start, size), :], wo,
                (((1,), (0,)), ((), ())),
                preferred_element_type=F32).astype(BF16)

        rs1 = rs2 = None
        for j in range(B):
            row = blocks[j] * SQ
            for h in range(H):
                n = j * H + h
                slot = n & 3
                if n + DEPTH < B * H:
                    nd = kv_copies(n + DEPTH, (n + DEPTH) & 3)
                    nd[0].start()
                    nd[1].start()
                    inflight[n + DEPTH] = nd
                ck, cv = inflight.pop(n)
                ck.wait()
                cv.wait()
                k = kbuf[slot].astype(BF16)
                v = vbuf[slot].astype(BF16)
                q = q_ref[pl.ds(row, SQ), h * DH:(h + 1) * DH]
                s = lax.dot_general(q, k, (((1,), (1,)), ((), ())),
                                    preferred_element_type=F32)
                e = jnp.exp(s)
                l = jnp.sum(e, axis=-1, keepdims=True)
                o = lax.dot_general(e.astype(BF16), v,
                                    (((1,), (0,)), ((), ())),
                                    preferred_element_type=F32)
                attn_ref[pl.ds(row, SQ), h * DH:(h + 1) * DH] = (
                    (o / l).astype(BF16))
            if not COMM:
                continue
            if j == 1:
                project(send1, 512)
                rs1 = exchange(0, 4, out_ref.at[pl.ds(send1, 512)],
                               recv_ref.at[0, pl.ds(0, 512)])
                rs1.start()
            elif j == 2:
                project(j2_start, 256)
                rs1.wait_recv()
                out_ref[pl.ds(j2_start, 256)] = (
                    out_ref[pl.ds(j2_start, 256)].astype(F32)
                    + recv_ref[0, pl.ds(256 * (1 - b1), 256)].astype(F32)
                ).astype(BF16)
                rs2 = exchange(1, 3, out_ref.at[pl.ds(j2_start, 256)],
                               recv_ref.at[1, pl.ds(0, 256)])
                rs2.start()

        if not COMM:
            project(a1, 512)
            project(send1, 512)
            return

        project(a2, 256)
        rs2.wait_recv()
        out_ref[pl.ds(a2, 256)] = (
            out_ref[pl.ds(a2, 256)].astype(F32)
            + recv_ref[0, pl.ds(256 * b1, 256)].astype(F32)
            + recv_ref[1, pl.ds(0, 256)].astype(F32)
        ).astype(BF16)

        rs3 = exchange(2, 1, out_ref.at[pl.ds(a2 + 128 * (1 - b0), 128)],
                       recv_ref.at[1, pl.ds(256, 128)])
        rs3.start()
        rs3.wait_recv()
        out_ref[pl.ds(a3, 128)] = (
            out_ref[pl.ds(a3, 128)].astype(F32)
            + recv_ref[1, pl.ds(256, 128)].astype(F32)
        ).astype(BF16)

        sends = []
        for m in (6, 2, 5, 7, 1, 3, 4):
            ag = exchange(2 + m, m, out_ref.at[pl.ds(a3, 128)],
                          out_ref.at[pl.ds(a3, 128)])
            ag.start()
            sends.append(ag)
        for m in (1, 3, 4, 2, 5, 7, 6):
            exchange(2 + m, m, out_ref.at[pl.ds(a3, 128)],
                     out_ref.at[pl.ds(128 * (p ^ m), 128)]).wait_recv()

        for r in (rs1, rs2, rs3, *sends):
            r.wait_send()

    out = pl.pallas_call(
        body,
        out_shape=jax.ShapeDtypeStruct((R, D), BF16),
        in_specs=[pl.BlockSpec(memory_space=pltpu.VMEM)] * 3
        + [pl.BlockSpec(memory_space=pl.ANY)] * 2,
        out_specs=pl.BlockSpec(memory_space=pltpu.VMEM),
        scratch_shapes=[
            pltpu.VMEM((R, D), BF16),
            pltpu.VMEM((R, D), BF16),
            pltpu.VMEM((2, 512, D), BF16),
            pltpu.VMEM((4, SKV, DH), F32),
            pltpu.VMEM((4, SKV, DH), F32),
            pltpu.SemaphoreType.DMA((10,)),
            pltpu.SemaphoreType.DMA((10,)),
            pltpu.SemaphoreType.DMA((4,)),
            pltpu.SemaphoreType.DMA((4,)),
        ],
        compiler_params=_CompilerParams(
            vmem_limit_bytes=100 * 1024 * 1024,
            **({"collective_id": 0} if COMM else {})),
    )(x, Wq, Wo, K_ext, V_ext)
    return out.reshape(B, SQ, D).astype(F32)
